# baseline (device time: 20539 ns/iter reference)
import jax
import jax.numpy as jnp
from jax import lax
from jax.experimental import pallas as pl
from jax.experimental.pallas import tpu as pltpu


def kernel(t, W):
    m, k = t.shape
    _, n = W.shape

    def body(t_ref, w_ref, out_ref, comm_ref, send_sems, recv_sems):
        my = lax.axis_index("i")
        peer1 = my ^ 1
        peer2 = 3 - my

        barrier_sem = pltpu.get_barrier_semaphore()
        for nbr in (peer1, peer2):
            pl.semaphore_signal(
                barrier_sem, inc=1,
                device_id=(nbr,), device_id_type=pl.DeviceIdType.MESH,
            )
        pl.semaphore_wait(barrier_sem, 2)

        out_ref[...] = jnp.dot(
            t_ref[...], w_ref[...], preferred_element_type=jnp.float32
        )

        for stage, peer in enumerate((peer1, peer2)):
            rdma = pltpu.make_async_remote_copy(
                src_ref=out_ref,
                dst_ref=comm_ref.at[stage],
                send_sem=send_sems.at[stage],
                recv_sem=recv_sems.at[stage],
                device_id=(peer,),
                device_id_type=pl.DeviceIdType.MESH,
            )
            rdma.start()
            rdma.wait()
            out_ref[...] += comm_ref[stage]

    return pl.pallas_call(
        body,
        out_shape=jax.ShapeDtypeStruct((m, n), jnp.float32),
        in_specs=[
            pl.BlockSpec(memory_space=pltpu.VMEM),
            pl.BlockSpec(memory_space=pltpu.VMEM),
        ],
        out_specs=pl.BlockSpec(memory_space=pltpu.VMEM),
        scratch_shapes=[
            pltpu.VMEM((2, m, n), jnp.float32),
            pltpu.SemaphoreType.DMA((2,)),
            pltpu.SemaphoreType.DMA((2,)),
        ],
        compiler_params=pltpu.CompilerParams(collective_id=0),
    )(t, W)


# device time: 16288 ns/iter; 1.2610x vs baseline; 1.2610x over previous
import jax
import jax.numpy as jnp
from jax import lax
from jax.experimental import pallas as pl
from jax.experimental.pallas import tpu as pltpu


C = 4


def kernel(t, W):
    m, k = t.shape
    _, n = W.shape
    mc = m // C

    def body(t_ref, w_ref, out_ref, comm_ref, send_sems, recv_sems):
        my = lax.axis_index("i")
        peer1 = my ^ 1
        peer2 = 3 - my

        barrier_sem = pltpu.get_barrier_semaphore()
        for nbr in (peer1, peer2):
            pl.semaphore_signal(
                barrier_sem, inc=1,
                device_id=(nbr,), device_id_type=pl.DeviceIdType.MESH,
            )
        pl.semaphore_wait(barrier_sem, 2)

        def exchange(stage, peer, c):
            return pltpu.make_async_remote_copy(
                src_ref=out_ref.at[pl.ds(c * mc, mc), :],
                dst_ref=comm_ref.at[stage, pl.ds(c * mc, mc), :],
                send_sem=send_sems.at[stage, c],
                recv_sem=recv_sems.at[stage, c],
                device_id=(peer,),
                device_id_type=pl.DeviceIdType.MESH,
            )

        s1 = []
        for c in range(C):
            rows = pl.ds(c * mc, mc)
            out_ref[rows, :] = jnp.dot(
                t_ref[rows, :], w_ref[...], preferred_element_type=jnp.float32
            )
            r = exchange(0, peer1, c)
            r.start()
            s1.append(r)

        s2 = []
        for c in range(C):
            rows = pl.ds(c * mc, mc)
            s1[c].wait()
            out_ref[rows, :] += comm_ref[0, rows, :]
            r = exchange(1, peer2, c)
            r.start()
            s2.append(r)

        for c in range(C):
            rows = pl.ds(c * mc, mc)
            s2[c].wait()
            out_ref[rows, :] += comm_ref[1, rows, :]

    return pl.pallas_call(
        body,
        out_shape=jax.ShapeDtypeStruct((m, n), jnp.float32),
        in_specs=[
            pl.BlockSpec(memory_space=pltpu.VMEM),
            pl.BlockSpec(memory_space=pltpu.VMEM),
        ],
        out_specs=pl.BlockSpec(memory_space=pltpu.VMEM),
        scratch_shapes=[
            pltpu.VMEM((2, m, n), jnp.float32),
            pltpu.SemaphoreType.DMA((2, C)),
            pltpu.SemaphoreType.DMA((2, C)),
        ],
        compiler_params=pltpu.CompilerParams(collective_id=0),
    )(t, W)


# device time: 13610 ns/iter; 1.5091x vs baseline; 1.1968x over previous
import jax
import jax.numpy as jnp
from jax import lax
from jax.experimental import pallas as pl
from jax.experimental.pallas import tpu as pltpu


H = 2
C2 = 2


def kernel(t, W):
    m, k = t.shape
    _, n = W.shape
    mh = m // H
    mc = mh // C2

    def body(t_ref, w_ref, out_ref, comm_ref, send_sems, recv_sems):
        my = lax.axis_index("i")
        peer1 = my ^ 1
        peer2 = 3 - my
        peers = (peer1, peer2)

        barrier_sem = pltpu.get_barrier_semaphore()
        for nbr in peers:
            pl.semaphore_signal(
                barrier_sem, inc=1,
                device_id=(nbr,), device_id_type=pl.DeviceIdType.MESH,
            )

        chunks = [
            (h, c, h * mh + c * mc) for c in range(C2) for h in range(H)
        ]

        def exchange(stage, h, c, row, peer):
            return pltpu.make_async_remote_copy(
                src_ref=out_ref.at[pl.ds(row, mc), :],
                dst_ref=comm_ref.at[stage, pl.ds(row, mc), :],
                send_sem=send_sems.at[stage, h, c],
                recv_sem=recv_sems.at[stage, h, c],
                device_id=(peer,),
                device_id_type=pl.DeviceIdType.MESH,
            )

        s1 = {}
        for i, (h, c, row) in enumerate(chunks):
            rows = pl.ds(row, mc)
            out_ref[rows, :] = jnp.dot(
                t_ref[rows, :], w_ref[...], preferred_element_type=jnp.float32
            )
            if i == 0:
                pl.semaphore_wait(barrier_sem, 2)
            r = exchange(0, h, c, row, peers[h])
            r.start()
            s1[(h, c)] = r

        s2 = {}
        for h, c, row in chunks:
            rows = pl.ds(row, mc)
            s1[(h, c)].wait()
            out_ref[rows, :] += comm_ref[0, rows, :]
            r = exchange(1, h, c, row, peers[1 - h])
            r.start()
            s2[(h, c)] = r

        for h, c, row in chunks:
            rows = pl.ds(row, mc)
            s2[(h, c)].wait()
            out_ref[rows, :] += comm_ref[1, rows, :]

    return pl.pallas_call(
        body,
        out_shape=jax.ShapeDtypeStruct((m, n), jnp.float32),
        in_specs=[
            pl.BlockSpec(memory_space=pltpu.VMEM),
            pl.BlockSpec(memory_space=pltpu.VMEM),
        ],
        out_specs=pl.BlockSpec(memory_space=pltpu.VMEM),
        scratch_shapes=[
            pltpu.VMEM((2, m, n), jnp.float32),
            pltpu.SemaphoreType.DMA((2, H, C2)),
            pltpu.SemaphoreType.DMA((2, H, C2)),
        ],
        compiler_params=pltpu.CompilerParams(collective_id=0),
    )(t, W)
